# baseline (speedup 1.0000x reference)
import functools

import jax
import jax.numpy as jnp
from jax import lax
from jax.experimental import pallas as pl
from jax.experimental.pallas import tpu as pltpu
from jax.experimental.pallas import tpu_sc as plsc

_NC = 2
_NS = 16
_NW = _NC * _NS
_L = 16

_BATCH = 4
_SEQ = 2048
_HIDDEN = 64
_B = _BATCH * _SEQ
_BPW = _B // _NW
_CHUNK = 128
_NCHUNK = _BPW // _CHUNK
_NVREG = _HIDDEN // _L


def _hsum(v):
    dnums = lax.GatherDimensionNumbers(
        offset_dims=(), collapsed_slice_dims=(0,), start_index_map=(0,))
    for sh in (8, 4, 2, 1):
        perm = (jnp.arange(_L, dtype=jnp.int32) ^ sh)[:, None]
        v = v + lax.gather(v, perm, dnums, slice_sizes=(1,),
                           mode=lax.GatherScatterMode.PROMISE_IN_BOUNDS)
    return v


def _rsqrt(v):
    vi = lax.bitcast_convert_type(v, jnp.int32)
    yi = jnp.int32(0x5F3759DF) - lax.shift_right_logical(vi, 1)
    y = lax.bitcast_convert_type(yi, jnp.float32)
    for _ in range(3):
        y = y * (1.5 - 0.5 * v * y * y)
    return y


def _sc_body(ids_hbm, table_hbm, pos_hbm, gamma_hbm, beta_hbm, out_hbm,
             idx_v, rows_v, pos_v, gamma_v, beta_v, sem):
    wid = lax.axis_index("s") * _NC + lax.axis_index("c")
    base = wid * _BPW
    pos_base = (wid % (_SEQ // _BPW)) * _BPW

    pltpu.sync_copy(ids_hbm.at[wid], idx_v)
    pltpu.sync_copy(pos_hbm.at[pl.ds(pos_base, _BPW)], pos_v)
    pltpu.sync_copy(gamma_hbm, gamma_v)
    pltpu.sync_copy(beta_hbm, beta_v)

    copies = [
        pltpu.async_copy(
            table_hbm.at[idx_v.at[j]],
            rows_v.at[pl.ds(j * _CHUNK, _CHUNK)],
            sem,
        )
        for j in range(_NCHUNK)
    ]
    for c in copies:
        c.wait()

    gs = [gamma_v[pl.ds(j * _L, _L)] for j in range(_NVREG)]
    bs = [beta_v[pl.ds(j * _L, _L)] for j in range(_NVREG)]
    inv_h = jnp.float32(1.0 / _HIDDEN)

    def row(i, carry):
        xs = [
            rows_v[i, pl.ds(j * _L, _L)] + pos_v[i, pl.ds(j * _L, _L)]
            for j in range(_NVREG)
        ]
        mean = _hsum(xs[0] + xs[1] + xs[2] + xs[3]) * inv_h
        ds = [x - mean for x in xs]
        var = _hsum(ds[0] * ds[0] + ds[1] * ds[1]
                    + ds[2] * ds[2] + ds[3] * ds[3]) * inv_h
        rstd = _rsqrt(var + 1e-12)
        for j in range(_NVREG):
            rows_v[i, pl.ds(j * _L, _L)] = ds[j] * rstd * gs[j] + bs[j]
        return carry

    lax.fori_loop(0, _BPW, row, 0)

    pltpu.sync_copy(rows_v, out_hbm.at[pl.ds(base, _BPW)])


@jax.jit
def _embed_ln(ids, token_table, pos_table, gamma, beta):
    mesh = plsc.VectorSubcoreMesh(core_axis_name="c", subcore_axis_name="s")
    kern = functools.partial(
        pl.kernel,
        out_type=jax.ShapeDtypeStruct((_B, _HIDDEN), jnp.float32),
        mesh=mesh,
        scratch_types=[
            pltpu.VMEM((_NCHUNK, _CHUNK), jnp.int32),
            pltpu.VMEM((_BPW, _HIDDEN), jnp.float32),
            pltpu.VMEM((_BPW, _HIDDEN), jnp.float32),
            pltpu.VMEM((_HIDDEN,), jnp.float32),
            pltpu.VMEM((_HIDDEN,), jnp.float32),
            pltpu.SemaphoreType.DMA,
        ],
        compiler_params=pltpu.CompilerParams(use_tc_tiling_on_sc=False),
    )(_sc_body)
    return kern(ids, token_table, pos_table, gamma, beta)


def kernel(input_ids, token_table, pos_table, gamma, beta):
    ids = input_ids.astype(jnp.int32).reshape(_NW, _NCHUNK, _CHUNK)
    out = _embed_ln(ids, token_table, pos_table, gamma, beta)
    return out.reshape(_BATCH, _SEQ, _HIDDEN)


# speedup vs baseline: 4.2249x; 4.2249x over previous
import functools

import jax
import jax.numpy as jnp
from jax import lax
from jax.experimental import pallas as pl
from jax.experimental.pallas import tpu as pltpu
from jax.experimental.pallas import tpu_sc as plsc

_NC = 2
_NS = 16
_NW = _NC * _NS
_L = 16

_BATCH = 4
_SEQ = 2048
_HIDDEN = 64
_B = _BATCH * _SEQ
_BPW = _B // _NW
_NGRP = _BPW // _L
_HH = _HIDDEN // 2


def _splat(v, lane):
    dnums = lax.GatherDimensionNumbers(
        offset_dims=(), collapsed_slice_dims=(0,), start_index_map=(0,))
    idx = jnp.full((_L, 1), lane, dtype=jnp.int32)
    return lax.gather(v, idx, dnums, slice_sizes=(1,),
                      mode=lax.GatherScatterMode.PROMISE_IN_BOUNDS)


def _rsqrt(v):
    vi = lax.bitcast_convert_type(v, jnp.int32)
    yi = jnp.int32(0x5F3759DF) - lax.shift_right_logical(vi, 1)
    y = lax.bitcast_convert_type(yi, jnp.float32)
    for _ in range(2):
        y = y * (1.5 - 0.5 * v * y * y)
    return y


def _sc_body(ids_hbm, ttab_hbm, pos_hbm, gamma_hbm, beta_hbm, out_hbm,
             idx_v, x_v, pos_v, gamma_v, beta_v, bank, sem):
    wid = lax.axis_index("s") * _NC + lax.axis_index("c")
    base = wid * _BPW
    pos_base = (wid % (_SEQ // _BPW)) * _BPW

    pltpu.sync_copy(ids_hbm.at[wid], idx_v)
    pltpu.sync_copy(pos_hbm.at[:, pl.ds(pos_base, _BPW)], pos_v)
    pltpu.sync_copy(gamma_hbm, gamma_v)
    pltpu.sync_copy(beta_hbm, beta_v)

    lane_iota = lax.iota(jnp.int32, _L)

    def group(g, carry):
        lanes = pl.ds(g * _L, _L)
        vec = idx_v[0, lanes]
        mvec = lax.bitwise_and(vec, jnp.int32(127))
        for half in range(2):
            h0 = half * _HH
            copies = []
            for l in range(_L):
                q = lax.shift_right_logical(vec[l], 7)
                col = pl.multiple_of(q * 128, 128)
                copies.append(pltpu.make_async_copy(
                    ttab_hbm.at[pl.ds(h0, _HH), pl.ds(col, 128)],
                    bank.at[l],
                    sem,
                ))
            for c in copies:
                c.start()
            for c in copies:
                c.wait()
            for h in range(_HH):
                vals = plsc.load_gather(
                    bank, [lane_iota, jnp.full((_L,), h, jnp.int32), mvec])
                x_v[h0 + h, lanes] = vals
        return carry

    lax.fori_loop(0, _NGRP, group, 0)

    inv_h = jnp.float32(1.0 / _HIDDEN)
    gvec = [gamma_v[pl.ds(k * _L, _L)] for k in range(_HIDDEN // _L)]
    bvec = [beta_v[pl.ds(k * _L, _L)] for k in range(_HIDDEN // _L)]

    def tile(t, carry):
        lanes = pl.ds(t * _L, _L)
        s = x_v[0, lanes] + pos_v[0, lanes]
        ss = s * s
        for h in range(1, _HIDDEN):
            x = x_v[h, lanes] + pos_v[h, lanes]
            s = s + x
            ss = ss + x * x
        mean = s * inv_h
        var = ss * inv_h - mean * mean
        rstd = _rsqrt(var + 1e-12)
        for h in range(_HIDDEN):
            g = _splat(gvec[h // _L], h % _L)
            b = _splat(bvec[h // _L], h % _L)
            x = x_v[h, lanes] + pos_v[h, lanes]
            x_v[h, lanes] = (x - mean) * rstd * g + b
        return carry

    lax.fori_loop(0, _NGRP, tile, 0)

    pltpu.sync_copy(x_v, out_hbm.at[:, pl.ds(base, _BPW)])


@jax.jit
def _embed_ln(ids, ttab, post, gamma, beta):
    mesh = plsc.VectorSubcoreMesh(core_axis_name="c", subcore_axis_name="s")
    kern = functools.partial(
        pl.kernel,
        out_type=jax.ShapeDtypeStruct((_HIDDEN, _B), jnp.float32),
        mesh=mesh,
        scratch_types=[
            pltpu.VMEM((1, _BPW), jnp.int32),
            pltpu.VMEM((_HIDDEN, _BPW), jnp.float32),
            pltpu.VMEM((_HIDDEN, _BPW), jnp.float32),
            pltpu.VMEM((_HIDDEN,), jnp.float32),
            pltpu.VMEM((_HIDDEN,), jnp.float32),
            pltpu.VMEM((_L, _HH, 128), jnp.float32),
            pltpu.SemaphoreType.DMA,
        ],
        compiler_params=pltpu.CompilerParams(
            use_tc_tiling_on_sc=True, needs_layout_passes=False),
    )(_sc_body)
    return kern(ids, ttab, post, gamma, beta)


def kernel(input_ids, token_table, pos_table, gamma, beta):
    ids = input_ids.astype(jnp.int32).reshape(_NW, 1, _BPW)
    ttab = token_table.T
    post = pos_table.T
    out_t = _embed_ln(ids, ttab, post, gamma, beta)
    return out_t.T.reshape(_BATCH, _SEQ, _HIDDEN)
